# baseline (speedup 1.0000x reference)
import functools

import jax
import jax.numpy as jnp
from jax import lax
from jax.experimental import pallas as pl
from jax.experimental.pallas import tpu as pltpu
from jax.experimental.pallas import tpu_sc as plsc

_N = 10000
_E = 320000
_D = 128
_H = 128
_C = 16
_G = 512

_NC = 2
_NS = 16
_NW = _NC * _NS
_EPW = _E // _NW
_CH = 128
_NCHUNK = 2 * (-(-_EPW // (2 * _CH)))
_EPW_PAD = _NCHUNK * _CH
_ACC_ROWS = 10112
_ROWS_PER_SUB = _ACC_ROWS // _NS
_NPAIR = _NCHUNK // 2
_NOCT = _NPAIR // 4


def _sc_agg_body(x_hbm, idx_hbm, out_hbm, ring, g0, g1, acc, sem0, sem1, semi):
    c = lax.axis_index("c")
    s = lax.axis_index("s")
    wid = c * _NS + s

    pltpu.sync_copy(idx_hbm.at[wid, pl.ds(0, 6)], ring.at[pl.ds(0, 6)])
    pltpu.async_copy(idx_hbm.at[wid, pl.ds(6, 2)], ring.at[pl.ds(6, 2)], semi)

    zero = jnp.zeros((16,), jnp.float32)

    def _zrow(i, carry):
        for j in range(_D // 16):
            g1[i, pl.ds(j * 16, 16)] = zero
        return carry

    lax.fori_loop(0, _CH, _zrow, 0)

    def _zacc(t, carry):
        @pl.when(lax.rem(t, _NS) == s)
        def _():
            pltpu.sync_copy(g1, acc.at[pl.ds(t * _CH, _CH)])
        return carry

    lax.fori_loop(0, _ACC_ROWS // _CH, _zacc, 0)
    plsc.subcore_barrier()

    def _do_pair(g, k, wait_idx, issue, refill):
        if wait_idx:
            pltpu.make_async_copy(
                idx_hbm.at[wid, pl.ds(0, 2)], ring.at[pl.ds(0, 2)], semi
            ).wait()
        for b, (buf, sem) in enumerate(((g0, sem0), (g1, sem1))):
            slot = 2 * k + b
            pltpu.make_async_copy(x_hbm.at[ring.at[slot, 0]], buf, sem).wait()
            pltpu.sync_copy(buf, acc.at[ring.at[slot, 1]], add=True)
            if issue:
                pltpu.async_copy(x_hbm.at[ring.at[(slot + 2) % 8, 0]], buf, sem)
        if refill:
            pltpu.async_copy(
                idx_hbm.at[wid, pl.ds(g + 8, 2)],
                ring.at[pl.ds(2 * k, 2)],
                semi,
            )

    pltpu.async_copy(x_hbm.at[ring.at[0, 0]], g0, sem0)
    pltpu.async_copy(x_hbm.at[ring.at[1, 0]], g1, sem1)

    for k in range(4):
        _do_pair(2 * k, k, wait_idx=(k >= 2), issue=True, refill=True)

    def _octet(o, carry):
        for k in range(4):
            _do_pair(o * 8 + 2 * k, k, True, True, True)
        return carry

    lax.fori_loop(1, _NOCT - 1, _octet, 0)

    gt = (_NOCT - 1) * 8
    for k in range(4):
        _do_pair(gt + 2 * k, k, wait_idx=(k <= 2), issue=(k <= 2), refill=False)
    plsc.subcore_barrier()

    pltpu.sync_copy(
        acc.at[pl.ds(s * _ROWS_PER_SUB, _ROWS_PER_SUB)],
        out_hbm.at[c, pl.ds(s * _ROWS_PER_SUB, _ROWS_PER_SUB)],
    )


_sc_agg = pl.kernel(
    _sc_agg_body,
    out_type=jax.ShapeDtypeStruct((_NC, _ACC_ROWS, _D), jnp.float32),
    mesh=plsc.VectorSubcoreMesh(core_axis_name="c", subcore_axis_name="s"),
    scratch_types=[
        pltpu.VMEM((8, 2, _CH), jnp.int32),
        pltpu.VMEM((_CH, _D), jnp.float32),
        pltpu.VMEM((_CH, _D), jnp.float32),
        pltpu.VMEM_SHARED((_ACC_ROWS, _D), jnp.float32),
        pltpu.SemaphoreType.DMA,
        pltpu.SemaphoreType.DMA,
        pltpu.SemaphoreType.DMA,
    ],
)


def _dense_body(x_ref, p_ref, wa_ref, g_ref, b_ref, wb_ref, o_ref):
    z = x_ref[...] + p_ref[0, : _N, :] + p_ref[1, : _N, :]
    h = jnp.dot(z, wa_ref[...], preferred_element_type=jnp.float32)
    mu = jnp.mean(h, axis=0, keepdims=True)
    d = h - mu
    var = jnp.mean(d * d, axis=0, keepdims=True)
    hn = g_ref[...] * d * lax.rsqrt(var + 1e-5) + b_ref[...]
    a = jnp.maximum(hn, 0.0)
    o_ref[...] = jnp.maximum(
        jnp.dot(a, wb_ref[...], preferred_element_type=jnp.float32), 0.0
    )


def _dense_layer(x, parts, wa, g, b, wb):
    return pl.pallas_call(
        _dense_body,
        out_shape=jax.ShapeDtypeStruct((_N, _H), jnp.float32),
    )(x, parts, wa, g.reshape(1, _H), b.reshape(1, _H), wb)


_POOL_CHUNK = 1000


def _head_body(h_ref, batch_ref, wl1_ref, wl2_ref, bl2_ref, o_ref):
    sums = jnp.zeros((_G, _H), jnp.float32)
    cnt = jnp.zeros((_G, 1), jnp.float32)
    ones = jnp.ones((_POOL_CHUNK, 1), jnp.float32)
    for i in range(_N // _POOL_CHUNK):
        hb = h_ref[pl.ds(i * _POOL_CHUNK, _POOL_CHUNK), :]
        bb = batch_ref[pl.ds(i * _POOL_CHUNK, _POOL_CHUNK), :]
        gid = lax.broadcasted_iota(jnp.int32, (_POOL_CHUNK, _G), 1)
        onehot = (bb == gid).astype(jnp.float32)
        sums = sums + lax.dot_general(
            onehot, hb, (((0,), (0,)), ((), ())),
            preferred_element_type=jnp.float32,
        )
        cnt = cnt + lax.dot_general(
            onehot, ones, (((0,), (0,)), ((), ())),
            preferred_element_type=jnp.float32,
        )
    pooled = sums / jnp.clip(cnt, 1.0, None)
    t = jnp.maximum(
        jnp.dot(pooled, wl1_ref[...], preferred_element_type=jnp.float32), 0.0
    )
    logits = jnp.dot(t, wl2_ref[...], preferred_element_type=jnp.float32) + bl2_ref[...]
    m = jnp.max(logits, axis=1, keepdims=True)
    lse = jnp.log(jnp.sum(jnp.exp(logits - m), axis=1, keepdims=True)) + m
    o_ref[...] = logits - lse


def _head(h, batch_col, wl1, wl2, bl2):
    return pl.pallas_call(
        _head_body,
        out_shape=jax.ShapeDtypeStruct((_G, _C), jnp.float32),
    )(h, batch_col, wl1, wl2, bl2.reshape(1, _C))


@jax.jit
def kernel(x, edge_index, batch, W1a, g1, b1, W1b, W2a, g2, b2, W2b, W3a, g3, b3, W3b, Wl1, Wl2, bl2):
    pad = _EPW_PAD - _EPW
    src = jnp.concatenate(
        [edge_index[0].reshape(_NW, _EPW),
         jnp.zeros((_NW, pad), jnp.int32)], axis=1
    ).reshape(_NW, _NCHUNK, 1, _CH)
    dst = jnp.concatenate(
        [edge_index[1].reshape(_NW, _EPW),
         jnp.full((_NW, pad), _N, jnp.int32)], axis=1
    ).reshape(_NW, _NCHUNK, 1, _CH)
    idx = jnp.concatenate([src, dst], axis=2)

    h = x
    for wa, g, b, wb in ((W1a, g1, b1, W1b), (W2a, g2, b2, W2b), (W3a, g3, b3, W3b)):
        parts = _sc_agg(h, idx)
        h = _dense_layer(h, parts, wa, g, b, wb)

    return _head(h, batch.reshape(_N, 1), Wl1, Wl2, bl2)


# speedup vs baseline: 1.2810x; 1.2810x over previous
import functools

import jax
import jax.numpy as jnp
from jax import lax
from jax.experimental import pallas as pl
from jax.experimental.pallas import tpu as pltpu
from jax.experimental.pallas import tpu_sc as plsc

_N = 10000
_E = 320000
_D = 128
_H = 128
_C = 16
_G = 512

_NC = 2
_NS = 16
_NW = _NC * _NS
_EPW = _E // _NW
_CH = 128
_NCHUNK = -(-_EPW // _CH)
_EPW_PAD = _NCHUNK * _CH
_ACC_ROWS = _NCHUNK * _CH
_ROWS_PER_SUB = _ACC_ROWS // _NS


def _sc_agg_body(x_hbm, src_hbm, dst_hbm, out_hbm, src_v, dst_v, gbuf, acc, sem):
    c = lax.axis_index("c")
    s = lax.axis_index("s")
    wid = c * _NS + s

    pltpu.sync_copy(src_hbm.at[wid], src_v)
    pltpu.sync_copy(dst_hbm.at[wid], dst_v)

    zero = jnp.zeros((16,), jnp.float32)

    def _zrow(i, carry):
        for j in range(_D // 16):
            gbuf[i, pl.ds(j * 16, 16)] = zero
        return carry

    lax.fori_loop(0, _CH, _zrow, 0)

    def _zacc(t, carry):
        @pl.when(lax.rem(t, _NS) == s)
        def _():
            pltpu.sync_copy(gbuf, acc.at[pl.ds(t * _CH, _CH)])
        return carry

    lax.fori_loop(0, _ACC_ROWS // _CH, _zacc, 0)
    plsc.subcore_barrier()

    def _body(g, carry):
        pltpu.async_copy(x_hbm.at[src_v.at[g]], gbuf, sem).wait()
        pltpu.sync_copy(gbuf, acc.at[dst_v.at[g]], add=True)
        return carry

    lax.fori_loop(0, _NCHUNK, _body, 0)
    plsc.subcore_barrier()

    pltpu.sync_copy(
        acc.at[pl.ds(s * _ROWS_PER_SUB, _ROWS_PER_SUB)],
        out_hbm.at[c, pl.ds(s * _ROWS_PER_SUB, _ROWS_PER_SUB)],
    )


_sc_agg = pl.kernel(
    _sc_agg_body,
    out_type=jax.ShapeDtypeStruct((_NC, _ACC_ROWS, _D), jnp.float32),
    mesh=plsc.VectorSubcoreMesh(core_axis_name="c", subcore_axis_name="s"),
    scratch_types=[
        pltpu.VMEM((_NCHUNK, _CH), jnp.int32),
        pltpu.VMEM((_NCHUNK, _CH), jnp.int32),
        pltpu.VMEM((_CH, _D), jnp.float32),
        pltpu.VMEM_SHARED((_ACC_ROWS, _D), jnp.float32),
        pltpu.SemaphoreType.DMA,
    ],
)


def _dense_body(x_ref, p_ref, wa_ref, g_ref, b_ref, wb_ref, o_ref):
    z = x_ref[...] + p_ref[0, : _N, :] + p_ref[1, : _N, :]
    h = jnp.dot(z, wa_ref[...], preferred_element_type=jnp.float32)
    mu = jnp.mean(h, axis=0, keepdims=True)
    d = h - mu
    var = jnp.mean(d * d, axis=0, keepdims=True)
    hn = g_ref[...] * d * lax.rsqrt(var + 1e-5) + b_ref[...]
    a = jnp.maximum(hn, 0.0)
    o_ref[...] = jnp.maximum(
        jnp.dot(a, wb_ref[...], preferred_element_type=jnp.float32), 0.0
    )


def _dense_layer(x, parts, wa, g, b, wb):
    return pl.pallas_call(
        _dense_body,
        out_shape=jax.ShapeDtypeStruct((_N, _H), jnp.float32),
    )(x, parts, wa, g.reshape(1, _H), b.reshape(1, _H), wb)


_POOL_CHUNK = 1000


def _head_body(h_ref, batch_ref, wl1_ref, wl2_ref, bl2_ref, o_ref):
    sums = jnp.zeros((_G, _H), jnp.float32)
    cnt = jnp.zeros((_G, 1), jnp.float32)
    ones = jnp.ones((_POOL_CHUNK, 1), jnp.float32)
    for i in range(_N // _POOL_CHUNK):
        hb = h_ref[pl.ds(i * _POOL_CHUNK, _POOL_CHUNK), :]
        bb = batch_ref[pl.ds(i * _POOL_CHUNK, _POOL_CHUNK), :]
        gid = lax.broadcasted_iota(jnp.int32, (_POOL_CHUNK, _G), 1)
        onehot = (bb == gid).astype(jnp.float32)
        sums = sums + lax.dot_general(
            onehot, hb, (((0,), (0,)), ((), ())),
            preferred_element_type=jnp.float32,
        )
        cnt = cnt + lax.dot_general(
            onehot, ones, (((0,), (0,)), ((), ())),
            preferred_element_type=jnp.float32,
        )
    pooled = sums / jnp.clip(cnt, 1.0, None)
    t = jnp.maximum(
        jnp.dot(pooled, wl1_ref[...], preferred_element_type=jnp.float32), 0.0
    )
    logits = jnp.dot(t, wl2_ref[...], preferred_element_type=jnp.float32) + bl2_ref[...]
    m = jnp.max(logits, axis=1, keepdims=True)
    lse = jnp.log(jnp.sum(jnp.exp(logits - m), axis=1, keepdims=True)) + m
    o_ref[...] = logits - lse


def _head(h, batch_col, wl1, wl2, bl2):
    return pl.pallas_call(
        _head_body,
        out_shape=jax.ShapeDtypeStruct((_G, _C), jnp.float32),
    )(h, batch_col, wl1, wl2, bl2.reshape(1, _C))


@jax.jit
def kernel(x, edge_index, batch, W1a, g1, b1, W1b, W2a, g2, b2, W2b, W3a, g3, b3, W3b, Wl1, Wl2, bl2):
    pad = _EPW_PAD - _EPW
    src = jnp.concatenate(
        [edge_index[0].reshape(_NW, _EPW),
         jnp.zeros((_NW, pad), jnp.int32)], axis=1
    ).reshape(_NW, _NCHUNK, _CH)
    dst = jnp.concatenate(
        [edge_index[1].reshape(_NW, _EPW),
         jnp.full((_NW, pad), _N, jnp.int32)], axis=1
    ).reshape(_NW, _NCHUNK, _CH)

    h = x
    for wa, g, b, wb in ((W1a, g1, b1, W1b), (W2a, g2, b2, W2b), (W3a, g3, b3, W3b)):
        parts = _sc_agg(h, src, dst)
        h = _dense_layer(h, parts, wa, g, b, wb)

    return _head(h, batch.reshape(_N, 1), Wl1, Wl2, bl2)
